# baseline (speedup 1.0000x reference)
import functools

import jax
import jax.numpy as jnp
from jax import lax
from jax.experimental import pallas as pl
from jax.experimental.pallas import tpu as pltpu
from jax.experimental.pallas import tpu_sc as plsc

NEG = 0.01
EPS = 1e-5
NC, NS = 2, 16
NW = NC * NS
EB = 512
G = 64
WIDE = 128


def _leaky(v):
    return jnp.where(v >= 0, v, NEG * v)


def _mesh():
    return plsc.VectorSubcoreMesh(core_axis_name="c", subcore_axis_name="s",
                                  num_cores=NC, num_subcores=NS)


def _sc_gather(table, idx, npass):
    e_pad = idx.shape[0]
    per_w = e_pad // NW
    pw = per_w // npass

    @functools.partial(
        pl.kernel,
        out_type=jax.ShapeDtypeStruct((e_pad, WIDE), jnp.float32),
        mesh=_mesh(),
        scratch_types=[
            pltpu.VMEM((pw,), jnp.int32), pltpu.VMEM((pw,), jnp.int32),
            pltpu.VMEM((pw, WIDE), jnp.float32),
            pltpu.VMEM((pw, WIDE), jnp.float32),
            pltpu.SemaphoreType.DMA, pltpu.SemaphoreType.DMA,
            pltpu.SemaphoreType.DMA, pltpu.SemaphoreType.DMA,
            pltpu.SemaphoreType.DMA, pltpu.SemaphoreType.DMA,
        ],
    )
    def body(table_hbm, idx_hbm, out_hbm, idxv0, idxv1, rows0, rows1,
             si0, si1, sg0, sg1, so0, so1):
        wid = lax.axis_index("s") * NC + lax.axis_index("c")
        idxv = (idxv0, idxv1)
        rows = (rows0, rows1)
        sis, sgs, sos = (si0, si1), (sg0, sg1), (so0, so1)
        dsc_i = [None, None]
        dsc_g = [None, None]
        dsc_o = [None, None]

        def base(p):
            return wid * per_w + p * pw

        dsc_i[0] = pltpu.async_copy(idx_hbm.at[pl.ds(base(0), pw)],
                                    idxv[0], sis[0])
        for p in range(npass):
            b = p & 1
            if p + 1 < npass:
                dsc_i[1 - b] = pltpu.async_copy(
                    idx_hbm.at[pl.ds(base(p + 1), pw)], idxv[1 - b],
                    sis[1 - b])
            if dsc_o[b] is not None:
                dsc_o[b].wait()
            dsc_i[b].wait()
            dsc_g[b] = pltpu.async_copy(table_hbm.at[idxv[b]], rows[b],
                                        sgs[b])
            dsc_g[b].wait()
            dsc_o[b] = pltpu.async_copy(rows[b],
                                        out_hbm.at[pl.ds(base(p), pw)],
                                        sos[b])
        for d in dsc_o:
            if d is not None:
                d.wait()

    return body(table, idx)


def _sc_scatter_add(msg, idxc, zeros_tile, npass):
    e_pad = msg.shape[0]
    rpt = zeros_tile.shape[0]
    n_half = rpt * NS
    per_w = e_pad // NS
    pw = per_w // npass

    @functools.partial(
        pl.kernel,
        out_type=jax.ShapeDtypeStruct((NC, n_half, WIDE), jnp.float32),
        mesh=_mesh(),
        scratch_types=[
            pltpu.VMEM((pw,), jnp.int32), pltpu.VMEM((pw,), jnp.int32),
            pltpu.VMEM((pw, WIDE), jnp.float32),
            pltpu.VMEM((pw, WIDE), jnp.float32),
            pltpu.VMEM_SHARED((n_half, WIDE), jnp.float32),
            pltpu.SemaphoreType.DMA, pltpu.SemaphoreType.DMA,
            pltpu.SemaphoreType.DMA, pltpu.SemaphoreType.DMA,
            pltpu.SemaphoreType.DMA, pltpu.SemaphoreType.DMA,
        ],
    )
    def body(msg_hbm, idxc_hbm, zer_hbm, part_hbm,
             idxv0, idxv1, rows0, rows1, table,
             si0, si1, sm0, sm1, sa0, sa1):
        cid = lax.axis_index("c")
        sid = lax.axis_index("s")
        idxv = (idxv0, idxv1)
        rows = (rows0, rows1)
        sis, sms, sas = (si0, si1), (sm0, sm1), (sa0, sa1)
        dsc_i = [None, None]
        dsc_m = [None, None]
        dsc_a = [None, None]

        def stage(p, b):
            base = sid * per_w + p * pw
            dsc_i[b] = pltpu.async_copy(
                idxc_hbm.at[pl.ds(cid * e_pad + base, pw)], idxv[b], sis[b])
            dsc_m[b] = pltpu.async_copy(msg_hbm.at[pl.ds(base, pw)],
                                        rows[b], sms[b])

        pltpu.sync_copy(zer_hbm, table.at[pl.ds(sid * rpt, rpt)])
        plsc.subcore_barrier()
        stage(0, 0)
        for p in range(npass):
            b = p & 1
            if p + 1 < npass:
                if dsc_a[1 - b] is not None:
                    dsc_a[1 - b].wait()
                stage(p + 1, 1 - b)
            dsc_i[b].wait()
            dsc_m[b].wait()
            dsc_a[b] = pltpu.async_copy(rows[b], table.at[idxv[b]],
                                        sas[b], add=True)
        for d in dsc_a:
            if d is not None:
                d.wait()
        plsc.subcore_barrier()
        pltpu.sync_copy(table.at[pl.ds(sid * rpt, rpt)],
                        part_hbm.at[cid, pl.ds(sid * rpt, rpt)])

    return body(msg, idxc, zeros_tile)


def _tc_norm(h, lnw, lnb, rootw, convb, nb):
    n, d = h.shape

    def body(h_ref, lnw_ref, lnb_ref, rw_ref, cb_ref, z_ref, zr_ref):
        hb = h_ref[...]
        mu = jnp.mean(hb, axis=-1, keepdims=True)
        var = jnp.mean((hb - mu) ** 2, axis=-1, keepdims=True)
        z = (hb - mu) * jax.lax.rsqrt(var + EPS) * lnw_ref[...] + lnb_ref[...]
        z = _leaky(z)
        z_ref[...] = jnp.concatenate(
            [z, jnp.zeros((nb, WIDE - d), jnp.float32)], axis=1)
        zr_ref[...] = (jnp.dot(z, rw_ref[...],
                               preferred_element_type=jnp.float32)
                       + cb_ref[...])

    return pl.pallas_call(
        body,
        grid=(n // nb,),
        in_specs=[
            pl.BlockSpec((nb, d), lambda i: (i, 0)),
            pl.BlockSpec((1, d), lambda i: (0, 0)),
            pl.BlockSpec((1, d), lambda i: (0, 0)),
            pl.BlockSpec((d, d), lambda i: (0, 0)),
            pl.BlockSpec((1, d), lambda i: (0, 0)),
        ],
        out_specs=[
            pl.BlockSpec((nb, WIDE), lambda i: (i, 0)),
            pl.BlockSpec((nb, d), lambda i: (i, 0)),
        ],
        out_shape=[
            jax.ShapeDtypeStruct((n, WIDE), jnp.float32),
            jax.ShapeDtypeStruct((n, d), jnp.float32),
        ],
    )(h, lnw, lnb, rootw, convb)


def _tc_edge(ea, xj, w1t, b1, w2t, b2, rep, e_real):
    e_pad, de = ea.shape
    d = 32
    hdim = w1t.shape[1]
    dd = w2t.shape[1]

    def body(ea_ref, xj_ref, w1_ref, b1_ref, w2_ref, b2_ref, rep_ref,
             msg_ref):
        i = pl.program_id(0)
        hid = _leaky(jnp.dot(ea_ref[...], w1_ref[...],
                             preferred_element_type=jnp.float32) + b1_ref[...])
        w = (jnp.dot(hid, w2_ref[...], preferred_element_type=jnp.float32)
             + b2_ref[...])
        xjb = xj_ref[...][:, 0:d]
        xjr = jnp.dot(xjb, rep_ref[...],
                      preferred_element_type=jnp.float32)
        acc = w[:, 0:128] * xjr[:, 0:128]
        for j in range(1, dd // 128):
            acc += w[:, j * 128:(j + 1) * 128] * xjr[:, j * 128:(j + 1) * 128]
        m = (acc[:, 0:32] + acc[:, 32:64] + acc[:, 64:96] + acc[:, 96:128])
        m = jnp.concatenate(
            [m, jnp.ones((EB, 16), jnp.float32),
             jnp.zeros((EB, WIDE - d - 16), jnp.float32)], axis=1)
        rows = i * EB + lax.broadcasted_iota(jnp.int32, (EB, WIDE), 0)
        msg_ref[...] = jnp.where(rows < e_real, m, 0.0)

    return pl.pallas_call(
        body,
        grid=(e_pad // EB,),
        in_specs=[
            pl.BlockSpec((EB, de), lambda i: (i, 0)),
            pl.BlockSpec((EB, WIDE), lambda i: (i, 0)),
            pl.BlockSpec((de, hdim), lambda i: (0, 0)),
            pl.BlockSpec((1, hdim), lambda i: (0, 0)),
            pl.BlockSpec((hdim, dd), lambda i: (0, 0)),
            pl.BlockSpec((1, dd), lambda i: (0, 0)),
            pl.BlockSpec((d, dd), lambda i: (0, 0)),
        ],
        out_specs=pl.BlockSpec((EB, WIDE), lambda i: (i, 0)),
        out_shape=jax.ShapeDtypeStruct((e_pad, WIDE), jnp.float32),
    )(ea, xj, w1t, b1, w2t, b2, rep)


def _tc_update(h, parts, zr, deg, nb, per_range):
    n, d = h.shape
    first = deg is None

    def body(*refs):
        if first:
            h_ref, p_ref, zr_ref, hn_ref, deg_ref = refs
        else:
            h_ref, p_ref, zr_ref, dg_ref, hn_ref = refs
        s = p_ref[0]
        agg = s[:, 0:d]
        if first:
            degv = jnp.maximum(s[:, d:d + 1], 1.0)
            deg_ref[...] = jnp.broadcast_to(degv, (nb, d))
        else:
            degv = dg_ref[...][:, 0:1]
        hn_ref[...] = h_ref[...] + agg / degv + zr_ref[...]

    in_specs = [
        pl.BlockSpec((nb, d), lambda i: (i, 0)),
        pl.BlockSpec((1, nb, WIDE),
                     lambda i: (i // per_range, i % per_range, 0)),
        pl.BlockSpec((nb, d), lambda i: (i, 0)),
    ]
    inputs = [h, parts, zr]
    if first:
        out_specs = [pl.BlockSpec((nb, d), lambda i: (i, 0)),
                     pl.BlockSpec((nb, d), lambda i: (i, 0))]
        out_shape = [jax.ShapeDtypeStruct((n, d), jnp.float32),
                     jax.ShapeDtypeStruct((n, d), jnp.float32)]
    else:
        in_specs.append(pl.BlockSpec((nb, d), lambda i: (i, 0)))
        inputs.append(deg)
        out_specs = pl.BlockSpec((nb, d), lambda i: (i, 0))
        out_shape = jax.ShapeDtypeStruct((n, d), jnp.float32)

    return pl.pallas_call(
        body,
        grid=(n // nb,),
        in_specs=in_specs,
        out_specs=out_specs,
        out_shape=out_shape,
    )(*inputs)


def _tc_head(h, batch2, dwt, db, owt, ob, nb):
    n, d = h.shape
    grid = n // nb

    def body(h_ref, b_ref, dwt_ref, db_ref, owt_ref, ob_ref, out_ref,
             sums, cnt):
        i = pl.program_id(0)

        @pl.when(i == 0)
        def _init():
            sums[...] = jnp.zeros_like(sums)
            cnt[...] = jnp.zeros_like(cnt)

        q = (jnp.dot(h_ref[...], dwt_ref[...],
                     preferred_element_type=jnp.float32) + db_ref[...])
        gids = lax.broadcasted_iota(jnp.int32, (nb, G), 1)
        oh = jnp.where(b_ref[...] == gids, 1.0, 0.0)
        sums[...] += lax.dot_general(oh, q, (((0,), (0,)), ((), ())),
                                     preferred_element_type=jnp.float32)
        cnt[...] += lax.dot_general(oh, jnp.ones((nb, 8), jnp.float32),
                                    (((0,), (0,)), ((), ())),
                                    preferred_element_type=jnp.float32)

        @pl.when(i == grid - 1)
        def _fin():
            pooled = _leaky(sums[...] / jnp.maximum(cnt[...][:, 0:1], 1.0))
            out_ref[...] = (jnp.dot(pooled, owt_ref[...],
                                    preferred_element_type=jnp.float32)
                            + ob_ref[...])

    return pl.pallas_call(
        body,
        grid=(grid,),
        in_specs=[
            pl.BlockSpec((nb, d), lambda i: (i, 0)),
            pl.BlockSpec((nb, 1), lambda i: (i, 0)),
            pl.BlockSpec((d, d), lambda i: (0, 0)),
            pl.BlockSpec((1, d), lambda i: (0, 0)),
            pl.BlockSpec((d, d), lambda i: (0, 0)),
            pl.BlockSpec((1, d), lambda i: (0, 0)),
        ],
        out_specs=pl.BlockSpec((G, d), lambda i: (0, 0)),
        out_shape=jax.ShapeDtypeStruct((G, d), jnp.float32),
        scratch_shapes=[pltpu.VMEM((G, d), jnp.float32),
                        pltpu.VMEM((G, 8), jnp.float32)],
    )(h, batch2, dwt, db, owt, ob)


def kernel(x, edge_index, edge_attr, batch, e_w1, e_b1, e_w2, e_b2,
           root_w, conv_b, ln_w, ln_b, dense_w, dense_b, out_w, out_b):
    n, d = x.shape
    e = edge_index.shape[1]
    de = edge_attr.shape[1]
    nlayers = e_w1.shape[0]

    align = NW * 128
    step = max(align, EB)
    e_pad = -(-e // step) * step
    pad_e = e_pad - e

    n_split = n // 2
    nbu = n_split // 5
    per_range = n_split // nbu
    rpt = -(-(n_split + 8) // (NS * 8)) * 8
    n_half = rpt * NS
    trash = n_half - 8

    src2 = jnp.concatenate([edge_index[0], jnp.zeros((pad_e,), jnp.int32)])
    dst = jnp.concatenate([edge_index[1], jnp.zeros((pad_e,), jnp.int32)])
    idxc = jnp.concatenate([jnp.where(dst < n_split, dst, trash),
                            jnp.where(dst >= n_split, dst - n_split, trash)])
    ea_pad = jnp.concatenate(
        [edge_attr, jnp.zeros((pad_e, de), jnp.float32)], axis=0)
    zeros_tile = jnp.zeros((rpt, WIDE), jnp.float32)
    rep = jnp.kron(jnp.eye(d, dtype=jnp.float32),
                   jnp.ones((1, d), jnp.float32))

    w1t = jnp.transpose(e_w1, (0, 2, 1))
    w2t = jnp.transpose(e_w2, (0, 2, 1))
    b1r = e_b1[:, None, :]
    b2r = e_b2[:, None, :]
    lnw = ln_w[:, None, :]
    lnb = ln_b[:, None, :]
    cbr = conv_b[:, None, :]

    h = x
    deg = None
    for l in range(nlayers):
        z, zr = _tc_norm(h, lnw[l], lnb[l], root_w[l], cbr[l], 1000)
        xj = _sc_gather(z, src2, 8)
        msg = _tc_edge(ea_pad, xj, w1t[l], b1r[l], w2t[l], b2r[l], rep, e)
        parts = _sc_scatter_add(msg, idxc, zeros_tile, 20)
        if deg is None:
            h, deg = _tc_update(h, parts, zr, None, nbu, per_range)
        else:
            h = _tc_update(h, parts, zr, deg, nbu, per_range)

    return _tc_head(h, batch[:, None], dense_w.T, dense_b[None, :],
                    out_w.T, out_b[None, :], 1000)


# speedup vs baseline: 1.0054x; 1.0054x over previous
import functools

import jax
import jax.numpy as jnp
from jax import lax
from jax.experimental import pallas as pl
from jax.experimental.pallas import tpu as pltpu
from jax.experimental.pallas import tpu_sc as plsc

NEG = 0.01
EPS = 1e-5
NC, NS = 2, 16
NW = NC * NS
EB = 512
G = 64
WIDE = 128


def _leaky(v):
    return jnp.where(v >= 0, v, NEG * v)


def _mesh():
    return plsc.VectorSubcoreMesh(core_axis_name="c", subcore_axis_name="s",
                                  num_cores=NC, num_subcores=NS)


def _sc_gather(table, idx, npass):
    e_pad = idx.shape[0]
    per_w = e_pad // NW
    pw = per_w // npass

    @functools.partial(
        pl.kernel,
        out_type=jax.ShapeDtypeStruct((e_pad, WIDE), jnp.float32),
        mesh=_mesh(),
        scratch_types=[
            pltpu.VMEM((pw,), jnp.int32), pltpu.VMEM((pw,), jnp.int32),
            pltpu.VMEM((pw, WIDE), jnp.float32),
            pltpu.VMEM((pw, WIDE), jnp.float32),
            pltpu.SemaphoreType.DMA, pltpu.SemaphoreType.DMA,
            pltpu.SemaphoreType.DMA, pltpu.SemaphoreType.DMA,
            pltpu.SemaphoreType.DMA, pltpu.SemaphoreType.DMA,
        ],
    )
    def body(table_hbm, idx_hbm, out_hbm, idxv0, idxv1, rows0, rows1,
             si0, si1, sg0, sg1, so0, so1):
        wid = lax.axis_index("s") * NC + lax.axis_index("c")
        idxv = (idxv0, idxv1)
        rows = (rows0, rows1)
        sis, sgs, sos = (si0, si1), (sg0, sg1), (so0, so1)
        dsc_i = [None, None]
        dsc_g = [None, None]
        dsc_o = [None, None]

        def base(p):
            return wid * per_w + p * pw

        dsc_i[0] = pltpu.async_copy(idx_hbm.at[pl.ds(base(0), pw)],
                                    idxv[0], sis[0])
        for p in range(npass):
            b = p & 1
            if p + 1 < npass:
                dsc_i[1 - b] = pltpu.async_copy(
                    idx_hbm.at[pl.ds(base(p + 1), pw)], idxv[1 - b],
                    sis[1 - b])
            if dsc_o[b] is not None:
                dsc_o[b].wait()
            dsc_i[b].wait()
            dsc_g[b] = pltpu.async_copy(table_hbm.at[idxv[b]], rows[b],
                                        sgs[b])
            dsc_g[b].wait()
            dsc_o[b] = pltpu.async_copy(rows[b],
                                        out_hbm.at[pl.ds(base(p), pw)],
                                        sos[b])
        for d in dsc_o:
            if d is not None:
                d.wait()

    return body(table, idx)


def _sc_scatter_add(msg, idxc, zeros_tile, npass):
    e_pad = msg.shape[0]
    rpt = zeros_tile.shape[0]
    n_half = rpt * NS
    per_w = e_pad // NS
    pw = per_w // npass

    @functools.partial(
        pl.kernel,
        out_type=jax.ShapeDtypeStruct((NC, n_half, WIDE), jnp.float32),
        mesh=_mesh(),
        scratch_types=[
            pltpu.VMEM((pw,), jnp.int32), pltpu.VMEM((pw,), jnp.int32),
            pltpu.VMEM((pw, WIDE), jnp.float32),
            pltpu.VMEM((pw, WIDE), jnp.float32),
            pltpu.VMEM_SHARED((n_half, WIDE), jnp.float32),
            pltpu.SemaphoreType.DMA, pltpu.SemaphoreType.DMA,
            pltpu.SemaphoreType.DMA, pltpu.SemaphoreType.DMA,
            pltpu.SemaphoreType.DMA, pltpu.SemaphoreType.DMA,
        ],
    )
    def body(msg_hbm, idxc_hbm, zer_hbm, part_hbm,
             idxv0, idxv1, rows0, rows1, table,
             si0, si1, sm0, sm1, sa0, sa1):
        cid = lax.axis_index("c")
        sid = lax.axis_index("s")
        idxv = (idxv0, idxv1)
        rows = (rows0, rows1)
        sis, sms, sas = (si0, si1), (sm0, sm1), (sa0, sa1)
        dsc_i = [None, None]
        dsc_m = [None, None]
        dsc_a = [None, None]

        def stage(p, b):
            base = sid * per_w + p * pw
            dsc_i[b] = pltpu.async_copy(
                idxc_hbm.at[pl.ds(cid * e_pad + base, pw)], idxv[b], sis[b])
            dsc_m[b] = pltpu.async_copy(msg_hbm.at[pl.ds(base, pw)],
                                        rows[b], sms[b])

        pltpu.sync_copy(zer_hbm, table.at[pl.ds(sid * rpt, rpt)])
        plsc.subcore_barrier()
        stage(0, 0)
        for p in range(npass):
            b = p & 1
            if p + 1 < npass:
                if dsc_a[1 - b] is not None:
                    dsc_a[1 - b].wait()
                stage(p + 1, 1 - b)
            dsc_i[b].wait()
            dsc_m[b].wait()
            dsc_a[b] = pltpu.async_copy(rows[b], table.at[idxv[b]],
                                        sas[b], add=True)
        for d in dsc_a:
            if d is not None:
                d.wait()
        plsc.subcore_barrier()
        pltpu.sync_copy(table.at[pl.ds(sid * rpt, rpt)],
                        part_hbm.at[cid, pl.ds(sid * rpt, rpt)])

    return body(msg, idxc, zeros_tile)


def _tc_norm(h, lnw, lnb, rootw, convb, nb):
    n, d = h.shape

    def body(h_ref, lnw_ref, lnb_ref, rw_ref, cb_ref, z_ref, zr_ref):
        hb = h_ref[...]
        mu = jnp.mean(hb, axis=-1, keepdims=True)
        var = jnp.mean((hb - mu) ** 2, axis=-1, keepdims=True)
        z = (hb - mu) * jax.lax.rsqrt(var + EPS) * lnw_ref[...] + lnb_ref[...]
        z = _leaky(z)
        z_ref[...] = jnp.concatenate(
            [z, jnp.zeros((nb, WIDE - d), jnp.float32)], axis=1)
        zr_ref[...] = (jnp.dot(z, rw_ref[...],
                               preferred_element_type=jnp.float32)
                       + cb_ref[...])

    return pl.pallas_call(
        body,
        grid=(n // nb,),
        in_specs=[
            pl.BlockSpec((nb, d), lambda i: (i, 0)),
            pl.BlockSpec((1, d), lambda i: (0, 0)),
            pl.BlockSpec((1, d), lambda i: (0, 0)),
            pl.BlockSpec((d, d), lambda i: (0, 0)),
            pl.BlockSpec((1, d), lambda i: (0, 0)),
        ],
        out_specs=[
            pl.BlockSpec((nb, WIDE), lambda i: (i, 0)),
            pl.BlockSpec((nb, d), lambda i: (i, 0)),
        ],
        out_shape=[
            jax.ShapeDtypeStruct((n, WIDE), jnp.float32),
            jax.ShapeDtypeStruct((n, d), jnp.float32),
        ],
    )(h, lnw, lnb, rootw, convb)


def _tc_edge(ea, xj, w1t, b1, w2t, b2, rep, e_real):
    e_pad, de = ea.shape
    d = 32
    hdim = w1t.shape[1]
    dd = w2t.shape[1]

    def body(ea_ref, xj_ref, w1_ref, b1_ref, w2_ref, b2_ref, rep_ref,
             msg_ref):
        i = pl.program_id(0)
        hid = _leaky(jnp.dot(ea_ref[...], w1_ref[...],
                             preferred_element_type=jnp.float32) + b1_ref[...])
        w = (jnp.dot(hid.astype(jnp.bfloat16), w2_ref[...],
                     preferred_element_type=jnp.float32)
             + b2_ref[...])
        xjb = xj_ref[...][:, 0:d]
        xjr = jnp.dot(xjb.astype(jnp.bfloat16), rep_ref[...],
                      preferred_element_type=jnp.float32)
        acc = w[:, 0:128] * xjr[:, 0:128]
        for j in range(1, dd // 128):
            acc += w[:, j * 128:(j + 1) * 128] * xjr[:, j * 128:(j + 1) * 128]
        m = (acc[:, 0:32] + acc[:, 32:64] + acc[:, 64:96] + acc[:, 96:128])
        m = jnp.concatenate(
            [m, jnp.ones((EB, 16), jnp.float32),
             jnp.zeros((EB, WIDE - d - 16), jnp.float32)], axis=1)
        rows = i * EB + lax.broadcasted_iota(jnp.int32, (EB, WIDE), 0)
        msg_ref[...] = jnp.where(rows < e_real, m, 0.0)

    return pl.pallas_call(
        body,
        grid=(e_pad // EB,),
        in_specs=[
            pl.BlockSpec((EB, de), lambda i: (i, 0)),
            pl.BlockSpec((EB, WIDE), lambda i: (i, 0)),
            pl.BlockSpec((de, hdim), lambda i: (0, 0)),
            pl.BlockSpec((1, hdim), lambda i: (0, 0)),
            pl.BlockSpec((hdim, dd), lambda i: (0, 0)),
            pl.BlockSpec((1, dd), lambda i: (0, 0)),
            pl.BlockSpec((d, dd), lambda i: (0, 0)),
        ],
        out_specs=pl.BlockSpec((EB, WIDE), lambda i: (i, 0)),
        out_shape=jax.ShapeDtypeStruct((e_pad, WIDE), jnp.float32),
    )(ea, xj, w1t, b1, w2t, b2, rep)


def _tc_update(h, parts, zr, deg, nb, per_range):
    n, d = h.shape
    first = deg is None

    def body(*refs):
        if first:
            h_ref, p_ref, zr_ref, hn_ref, deg_ref = refs
        else:
            h_ref, p_ref, zr_ref, dg_ref, hn_ref = refs
        s = p_ref[0]
        agg = s[:, 0:d]
        if first:
            degv = jnp.maximum(s[:, d:d + 1], 1.0)
            deg_ref[...] = jnp.broadcast_to(degv, (nb, d))
        else:
            degv = dg_ref[...][:, 0:1]
        hn_ref[...] = h_ref[...] + agg / degv + zr_ref[...]

    in_specs = [
        pl.BlockSpec((nb, d), lambda i: (i, 0)),
        pl.BlockSpec((1, nb, WIDE),
                     lambda i: (i // per_range, i % per_range, 0)),
        pl.BlockSpec((nb, d), lambda i: (i, 0)),
    ]
    inputs = [h, parts, zr]
    if first:
        out_specs = [pl.BlockSpec((nb, d), lambda i: (i, 0)),
                     pl.BlockSpec((nb, d), lambda i: (i, 0))]
        out_shape = [jax.ShapeDtypeStruct((n, d), jnp.float32),
                     jax.ShapeDtypeStruct((n, d), jnp.float32)]
    else:
        in_specs.append(pl.BlockSpec((nb, d), lambda i: (i, 0)))
        inputs.append(deg)
        out_specs = pl.BlockSpec((nb, d), lambda i: (i, 0))
        out_shape = jax.ShapeDtypeStruct((n, d), jnp.float32)

    return pl.pallas_call(
        body,
        grid=(n // nb,),
        in_specs=in_specs,
        out_specs=out_specs,
        out_shape=out_shape,
    )(*inputs)


def _tc_head(h, batch2, dwt, db, owt, ob, nb):
    n, d = h.shape
    grid = n // nb

    def body(h_ref, b_ref, dwt_ref, db_ref, owt_ref, ob_ref, out_ref,
             sums, cnt):
        i = pl.program_id(0)

        @pl.when(i == 0)
        def _init():
            sums[...] = jnp.zeros_like(sums)
            cnt[...] = jnp.zeros_like(cnt)

        q = (jnp.dot(h_ref[...], dwt_ref[...],
                     preferred_element_type=jnp.float32) + db_ref[...])
        gids = lax.broadcasted_iota(jnp.int32, (nb, G), 1)
        oh = jnp.where(b_ref[...] == gids, 1.0, 0.0)
        sums[...] += lax.dot_general(oh, q, (((0,), (0,)), ((), ())),
                                     preferred_element_type=jnp.float32)
        cnt[...] += lax.dot_general(oh, jnp.ones((nb, 8), jnp.float32),
                                    (((0,), (0,)), ((), ())),
                                    preferred_element_type=jnp.float32)

        @pl.when(i == grid - 1)
        def _fin():
            pooled = _leaky(sums[...] / jnp.maximum(cnt[...][:, 0:1], 1.0))
            out_ref[...] = (jnp.dot(pooled, owt_ref[...],
                                    preferred_element_type=jnp.float32)
                            + ob_ref[...])

    return pl.pallas_call(
        body,
        grid=(grid,),
        in_specs=[
            pl.BlockSpec((nb, d), lambda i: (i, 0)),
            pl.BlockSpec((nb, 1), lambda i: (i, 0)),
            pl.BlockSpec((d, d), lambda i: (0, 0)),
            pl.BlockSpec((1, d), lambda i: (0, 0)),
            pl.BlockSpec((d, d), lambda i: (0, 0)),
            pl.BlockSpec((1, d), lambda i: (0, 0)),
        ],
        out_specs=pl.BlockSpec((G, d), lambda i: (0, 0)),
        out_shape=jax.ShapeDtypeStruct((G, d), jnp.float32),
        scratch_shapes=[pltpu.VMEM((G, d), jnp.float32),
                        pltpu.VMEM((G, 8), jnp.float32)],
    )(h, batch2, dwt, db, owt, ob)


def kernel(x, edge_index, edge_attr, batch, e_w1, e_b1, e_w2, e_b2,
           root_w, conv_b, ln_w, ln_b, dense_w, dense_b, out_w, out_b):
    n, d = x.shape
    e = edge_index.shape[1]
    de = edge_attr.shape[1]
    nlayers = e_w1.shape[0]

    align = NW * 128
    step = max(align, EB)
    e_pad = -(-e // step) * step
    pad_e = e_pad - e

    n_split = n // 2
    nbu = n_split // 5
    per_range = n_split // nbu
    rpt = -(-(n_split + 8) // (NS * 8)) * 8
    n_half = rpt * NS
    trash = n_half - 8

    src2 = jnp.concatenate([edge_index[0], jnp.zeros((pad_e,), jnp.int32)])
    dst = jnp.concatenate([edge_index[1], jnp.zeros((pad_e,), jnp.int32)])
    idxc = jnp.concatenate([jnp.where(dst < n_split, dst, trash),
                            jnp.where(dst >= n_split, dst - n_split, trash)])
    ea_pad = jnp.concatenate(
        [edge_attr, jnp.zeros((pad_e, de), jnp.float32)], axis=0)
    zeros_tile = jnp.zeros((rpt, WIDE), jnp.float32)
    rep = jnp.kron(jnp.eye(d, dtype=jnp.float32),
                   jnp.ones((1, d), jnp.float32)).astype(jnp.bfloat16)

    w1t = jnp.transpose(e_w1, (0, 2, 1))
    w2t = jnp.transpose(e_w2, (0, 2, 1)).astype(jnp.bfloat16)
    b1r = e_b1[:, None, :]
    b2r = e_b2[:, None, :]
    lnw = ln_w[:, None, :]
    lnb = ln_b[:, None, :]
    cbr = conv_b[:, None, :]

    h = x
    deg = None
    for l in range(nlayers):
        z, zr = _tc_norm(h, lnw[l], lnb[l], root_w[l], cbr[l], 1000)
        xj = _sc_gather(z, src2, 8)
        msg = _tc_edge(ea_pad, xj, w1t[l], b1r[l], w2t[l], b2r[l], rep, e)
        parts = _sc_scatter_add(msg, idxc, zeros_tile, 20)
        if deg is None:
            h, deg = _tc_update(h, parts, zr, None, nbu, per_range)
        else:
            h = _tc_update(h, parts, zr, deg, nbu, per_range)

    return _tc_head(h, batch[:, None], dense_w.T, dense_b[None, :],
                    out_w.T, out_b[None, :], 1000)


# speedup vs baseline: 1.2488x; 1.2422x over previous
import functools

import jax
import jax.numpy as jnp
from jax import lax
from jax.experimental import pallas as pl
from jax.experimental.pallas import tpu as pltpu
from jax.experimental.pallas import tpu_sc as plsc

NEG = 0.01
EPS = 1e-5
NC, NS = 2, 16
NW = NC * NS
EB = 512
G = 64
WIDE = 128


def _leaky(v):
    return jnp.where(v >= 0, v, NEG * v)


def _mesh():
    return plsc.VectorSubcoreMesh(core_axis_name="c", subcore_axis_name="s",
                                  num_cores=NC, num_subcores=NS)


def _sc_gather(table, idx, npass, e0, e_len):
    per_w = e_len // NW
    pw = per_w // npass

    @functools.partial(
        pl.kernel,
        out_type=jax.ShapeDtypeStruct((e_len, WIDE), jnp.float32),
        mesh=_mesh(),
        scratch_types=[
            pltpu.VMEM((pw,), jnp.int32), pltpu.VMEM((pw,), jnp.int32),
            pltpu.VMEM((pw, WIDE), jnp.float32),
            pltpu.VMEM((pw, WIDE), jnp.float32),
            pltpu.SemaphoreType.DMA, pltpu.SemaphoreType.DMA,
            pltpu.SemaphoreType.DMA, pltpu.SemaphoreType.DMA,
            pltpu.SemaphoreType.DMA, pltpu.SemaphoreType.DMA,
        ],
    )
    def body(table_hbm, idx_hbm, out_hbm, idxv0, idxv1, rows0, rows1,
             si0, si1, sg0, sg1, so0, so1):
        wid = lax.axis_index("s") * NC + lax.axis_index("c")
        idxv = (idxv0, idxv1)
        rows = (rows0, rows1)
        sis, sgs, sos = (si0, si1), (sg0, sg1), (so0, so1)
        dsc_i = [None, None]
        dsc_g = [None, None]
        dsc_o = [None, None]

        def base(p):
            return wid * per_w + p * pw

        dsc_i[0] = pltpu.async_copy(idx_hbm.at[pl.ds(e0 + base(0), pw)],
                                    idxv[0], sis[0])
        for p in range(npass):
            b = p & 1
            if p + 1 < npass:
                dsc_i[1 - b] = pltpu.async_copy(
                    idx_hbm.at[pl.ds(e0 + base(p + 1), pw)], idxv[1 - b],
                    sis[1 - b])
            if dsc_o[b] is not None:
                dsc_o[b].wait()
            dsc_i[b].wait()
            dsc_g[b] = pltpu.async_copy(table_hbm.at[idxv[b]], rows[b],
                                        sgs[b])
            dsc_g[b].wait()
            dsc_o[b] = pltpu.async_copy(rows[b],
                                        out_hbm.at[pl.ds(base(p), pw)],
                                        sos[b])
        for d in dsc_o:
            if d is not None:
                d.wait()

    return body(table, idx)


def _sc_scatter_add(msg, idxc, zeros_tile, npass, e0, e_tot):
    e_len = msg.shape[0]
    rpt = zeros_tile.shape[0]
    n_half = rpt * NS
    per_w = e_len // NS
    pw = per_w // npass

    @functools.partial(
        pl.kernel,
        out_type=jax.ShapeDtypeStruct((NC, n_half, WIDE), jnp.float32),
        mesh=_mesh(),
        scratch_types=[
            pltpu.VMEM((pw,), jnp.int32), pltpu.VMEM((pw,), jnp.int32),
            pltpu.VMEM((pw, WIDE), jnp.float32),
            pltpu.VMEM((pw, WIDE), jnp.float32),
            pltpu.VMEM_SHARED((n_half, WIDE), jnp.float32),
            pltpu.SemaphoreType.DMA, pltpu.SemaphoreType.DMA,
            pltpu.SemaphoreType.DMA, pltpu.SemaphoreType.DMA,
            pltpu.SemaphoreType.DMA, pltpu.SemaphoreType.DMA,
        ],
    )
    def body(msg_hbm, idxc_hbm, zer_hbm, part_hbm,
             idxv0, idxv1, rows0, rows1, table,
             si0, si1, sm0, sm1, sa0, sa1):
        cid = lax.axis_index("c")
        sid = lax.axis_index("s")
        idxv = (idxv0, idxv1)
        rows = (rows0, rows1)
        sis, sms, sas = (si0, si1), (sm0, sm1), (sa0, sa1)
        dsc_i = [None, None]
        dsc_m = [None, None]
        dsc_a = [None, None]

        def stage(p, b):
            base = sid * per_w + p * pw
            dsc_i[b] = pltpu.async_copy(
                idxc_hbm.at[pl.ds(cid * e_tot + e0 + base, pw)],
                idxv[b], sis[b])
            dsc_m[b] = pltpu.async_copy(msg_hbm.at[pl.ds(base, pw)],
                                        rows[b], sms[b])

        pltpu.sync_copy(zer_hbm, table.at[pl.ds(sid * rpt, rpt)])
        plsc.subcore_barrier()
        stage(0, 0)
        for p in range(npass):
            b = p & 1
            if p + 1 < npass:
                if dsc_a[1 - b] is not None:
                    dsc_a[1 - b].wait()
                stage(p + 1, 1 - b)
            dsc_i[b].wait()
            dsc_m[b].wait()
            dsc_a[b] = pltpu.async_copy(rows[b], table.at[idxv[b]],
                                        sas[b], add=True)
        for d in dsc_a:
            if d is not None:
                d.wait()
        plsc.subcore_barrier()
        pltpu.sync_copy(table.at[pl.ds(sid * rpt, rpt)],
                        part_hbm.at[cid, pl.ds(sid * rpt, rpt)])

    return body(msg, idxc, zeros_tile)


def _tc_norm(h, lnw, lnb, rootw, convb, nb):
    n, d = h.shape

    def body(h_ref, lnw_ref, lnb_ref, rw_ref, cb_ref, z_ref, zr_ref):
        hb = h_ref[...]
        mu = jnp.mean(hb, axis=-1, keepdims=True)
        var = jnp.mean((hb - mu) ** 2, axis=-1, keepdims=True)
        z = (hb - mu) * jax.lax.rsqrt(var + EPS) * lnw_ref[...] + lnb_ref[...]
        z = _leaky(z)
        z_ref[...] = jnp.concatenate(
            [z, jnp.zeros((nb, WIDE - d), jnp.float32)], axis=1)
        zr_ref[...] = (jnp.dot(z, rw_ref[...],
                               preferred_element_type=jnp.float32)
                       + cb_ref[...])

    return pl.pallas_call(
        body,
        grid=(n // nb,),
        in_specs=[
            pl.BlockSpec((nb, d), lambda i: (i, 0)),
            pl.BlockSpec((1, d), lambda i: (0, 0)),
            pl.BlockSpec((1, d), lambda i: (0, 0)),
            pl.BlockSpec((d, d), lambda i: (0, 0)),
            pl.BlockSpec((1, d), lambda i: (0, 0)),
        ],
        out_specs=[
            pl.BlockSpec((nb, WIDE), lambda i: (i, 0)),
            pl.BlockSpec((nb, d), lambda i: (i, 0)),
        ],
        out_shape=[
            jax.ShapeDtypeStruct((n, WIDE), jnp.float32),
            jax.ShapeDtypeStruct((n, d), jnp.float32),
        ],
    )(h, lnw, lnb, rootw, convb)


def _tc_edge(ea, xj, w1t, b1, w2t, b2, rep, e_real, e0):
    de = ea.shape[1]
    e_len = xj.shape[0]
    d = 32
    hdim = w1t.shape[1]
    dd = w2t.shape[1]

    def body(ea_ref, xj_ref, w1_ref, b1_ref, w2_ref, b2_ref, rep_ref,
             msg_ref):
        i = pl.program_id(0)
        hid = _leaky(jnp.dot(ea_ref[...], w1_ref[...],
                             preferred_element_type=jnp.float32) + b1_ref[...])
        w = (jnp.dot(hid.astype(jnp.bfloat16), w2_ref[...],
                     preferred_element_type=jnp.float32)
             + b2_ref[...])
        xjb = xj_ref[...][:, 0:d]
        xjr = jnp.dot(xjb.astype(jnp.bfloat16), rep_ref[...],
                      preferred_element_type=jnp.float32)
        acc = w[:, 0:128] * xjr[:, 0:128]
        for j in range(1, dd // 128):
            acc += w[:, j * 128:(j + 1) * 128] * xjr[:, j * 128:(j + 1) * 128]
        m = (acc[:, 0:32] + acc[:, 32:64] + acc[:, 64:96] + acc[:, 96:128])
        m = jnp.concatenate(
            [m, jnp.ones((EB, 16), jnp.float32),
             jnp.zeros((EB, WIDE - d - 16), jnp.float32)], axis=1)
        rows = e0 + i * EB + lax.broadcasted_iota(jnp.int32, (EB, WIDE), 0)
        msg_ref[...] = jnp.where(rows < e_real, m, 0.0)

    return pl.pallas_call(
        body,
        grid=(e_len // EB,),
        in_specs=[
            pl.BlockSpec((EB, de), lambda i: (i + e0 // EB, 0)),
            pl.BlockSpec((EB, WIDE), lambda i: (i, 0)),
            pl.BlockSpec((de, hdim), lambda i: (0, 0)),
            pl.BlockSpec((1, hdim), lambda i: (0, 0)),
            pl.BlockSpec((hdim, dd), lambda i: (0, 0)),
            pl.BlockSpec((1, dd), lambda i: (0, 0)),
            pl.BlockSpec((d, dd), lambda i: (0, 0)),
        ],
        out_specs=pl.BlockSpec((EB, WIDE), lambda i: (i, 0)),
        out_shape=jax.ShapeDtypeStruct((e_len, WIDE), jnp.float32),
    )(ea, xj, w1t, b1, w2t, b2, rep)


def _tc_update(h, parts, parts2, zr, deg, nb, per_range):
    n, d = h.shape
    first = deg is None

    def body(*refs):
        if first:
            h_ref, p_ref, q_ref, zr_ref, hn_ref, deg_ref = refs
        else:
            h_ref, p_ref, q_ref, zr_ref, dg_ref, hn_ref = refs
        s = p_ref[0] + q_ref[0]
        agg = s[:, 0:d]
        if first:
            degv = jnp.maximum(s[:, d:d + 1], 1.0)
            deg_ref[...] = jnp.broadcast_to(degv, (nb, d))
        else:
            degv = dg_ref[...][:, 0:1]
        hn_ref[...] = h_ref[...] + agg / degv + zr_ref[...]

    in_specs = [
        pl.BlockSpec((nb, d), lambda i: (i, 0)),
        pl.BlockSpec((1, nb, WIDE),
                     lambda i: (i // per_range, i % per_range, 0)),
        pl.BlockSpec((1, nb, WIDE),
                     lambda i: (i // per_range, i % per_range, 0)),
        pl.BlockSpec((nb, d), lambda i: (i, 0)),
    ]
    inputs = [h, parts, parts2, zr]
    if first:
        out_specs = [pl.BlockSpec((nb, d), lambda i: (i, 0)),
                     pl.BlockSpec((nb, d), lambda i: (i, 0))]
        out_shape = [jax.ShapeDtypeStruct((n, d), jnp.float32),
                     jax.ShapeDtypeStruct((n, d), jnp.float32)]
    else:
        in_specs.append(pl.BlockSpec((nb, d), lambda i: (i, 0)))
        inputs.append(deg)
        out_specs = pl.BlockSpec((nb, d), lambda i: (i, 0))
        out_shape = jax.ShapeDtypeStruct((n, d), jnp.float32)

    return pl.pallas_call(
        body,
        grid=(n // nb,),
        in_specs=in_specs,
        out_specs=out_specs,
        out_shape=out_shape,
    )(*inputs)


def _tc_head(h, batch2, dwt, db, owt, ob, nb):
    n, d = h.shape
    grid = n // nb

    def body(h_ref, b_ref, dwt_ref, db_ref, owt_ref, ob_ref, out_ref,
             sums, cnt):
        i = pl.program_id(0)

        @pl.when(i == 0)
        def _init():
            sums[...] = jnp.zeros_like(sums)
            cnt[...] = jnp.zeros_like(cnt)

        q = (jnp.dot(h_ref[...], dwt_ref[...],
                     preferred_element_type=jnp.float32) + db_ref[...])
        gids = lax.broadcasted_iota(jnp.int32, (nb, G), 1)
        oh = jnp.where(b_ref[...] == gids, 1.0, 0.0)
        sums[...] += lax.dot_general(oh, q, (((0,), (0,)), ((), ())),
                                     preferred_element_type=jnp.float32)
        cnt[...] += lax.dot_general(oh, jnp.ones((nb, 8), jnp.float32),
                                    (((0,), (0,)), ((), ())),
                                    preferred_element_type=jnp.float32)

        @pl.when(i == grid - 1)
        def _fin():
            pooled = _leaky(sums[...] / jnp.maximum(cnt[...][:, 0:1], 1.0))
            out_ref[...] = (jnp.dot(pooled, owt_ref[...],
                                    preferred_element_type=jnp.float32)
                            + ob_ref[...])

    return pl.pallas_call(
        body,
        grid=(grid,),
        in_specs=[
            pl.BlockSpec((nb, d), lambda i: (i, 0)),
            pl.BlockSpec((nb, 1), lambda i: (i, 0)),
            pl.BlockSpec((d, d), lambda i: (0, 0)),
            pl.BlockSpec((1, d), lambda i: (0, 0)),
            pl.BlockSpec((d, d), lambda i: (0, 0)),
            pl.BlockSpec((1, d), lambda i: (0, 0)),
        ],
        out_specs=pl.BlockSpec((G, d), lambda i: (0, 0)),
        out_shape=jax.ShapeDtypeStruct((G, d), jnp.float32),
        scratch_shapes=[pltpu.VMEM((G, d), jnp.float32),
                        pltpu.VMEM((G, 8), jnp.float32)],
    )(h, batch2, dwt, db, owt, ob)


def kernel(x, edge_index, edge_attr, batch, e_w1, e_b1, e_w2, e_b2,
           root_w, conv_b, ln_w, ln_b, dense_w, dense_b, out_w, out_b):
    n, d = x.shape
    e = edge_index.shape[1]
    de = edge_attr.shape[1]
    nlayers = e_w1.shape[0]

    align = NW * 128
    step = max(align, EB)
    e_pad = -(-e // step) * step
    pad_e = e_pad - e

    n_split = n // 2
    nbu = n_split // 5
    per_range = n_split // nbu
    rpt = -(-(n_split + 8) // (NS * 8)) * 8
    n_half = rpt * NS
    trash = n_half - 8

    src2 = jnp.concatenate([edge_index[0], jnp.zeros((pad_e,), jnp.int32)])
    dst = jnp.concatenate([edge_index[1], jnp.zeros((pad_e,), jnp.int32)])
    idxc = jnp.concatenate([jnp.where(dst < n_split, dst, trash),
                            jnp.where(dst >= n_split, dst - n_split, trash)])
    ea_pad = jnp.concatenate(
        [edge_attr, jnp.zeros((pad_e, de), jnp.float32)], axis=0)
    zeros_tile = jnp.zeros((rpt, WIDE), jnp.float32)
    rep = jnp.kron(jnp.eye(d, dtype=jnp.float32),
                   jnp.ones((1, d), jnp.float32)).astype(jnp.bfloat16)

    w1t = jnp.transpose(e_w1, (0, 2, 1))
    w2t = jnp.transpose(e_w2, (0, 2, 1)).astype(jnp.bfloat16)
    b1r = e_b1[:, None, :]
    b2r = e_b2[:, None, :]
    lnw = ln_w[:, None, :]
    lnb = ln_b[:, None, :]
    cbr = conv_b[:, None, :]

    h = x
    deg = None
    half = e_pad // 2
    for l in range(nlayers):
        z, zr = _tc_norm(h, lnw[l], lnb[l], root_w[l], cbr[l], 1000)
        xj_a = _sc_gather(z, src2, 4, 0, half)
        msg_a = _tc_edge(ea_pad, xj_a, w1t[l], b1r[l], w2t[l], b2r[l],
                         rep, e, 0)
        xj_b = _sc_gather(z, src2, 4, half, half)
        msg_b = _tc_edge(ea_pad, xj_b, w1t[l], b1r[l], w2t[l], b2r[l],
                         rep, e, half)
        parts_a = _sc_scatter_add(msg_a, idxc, zeros_tile, 10, 0, e_pad)
        parts_b = _sc_scatter_add(msg_b, idxc, zeros_tile, 10, half, e_pad)
        if deg is None:
            h, deg = _tc_update(h, parts_a, parts_b, zr, None, nbu,
                                per_range)
        else:
            h = _tc_update(h, parts_a, parts_b, zr, deg, nbu, per_range)

    return _tc_head(h, batch[:, None], dense_w.T, dense_b[None, :],
                    out_w.T, out_b[None, :], 1000)
